# baseline (speedup 1.0000x reference)
import jax
import jax.numpy as jnp
from jax import lax
from jax.experimental import pallas as pl
from jax.experimental.pallas import tpu as pltpu
from jax.experimental.pallas import tpu_sc as plsc

VOCAB = 100000
D_MODEL = 768
SEQ_LEN = 2048
BATCH = 4

NC = 2
NS = 16
LANES = 16
NW = NC * NS

S_PER_W = SEQ_LEN // NW
CHUNK = 32
N_K = S_PER_W // CHUNK
N_CHUNKS = N_K * BATCH
NBUF = 3
VECS_PER_ROW = D_MODEL // LANES


def _body(table_hbm, idx_hbm, pos_hbm, out_hbm,
          idx_v, pos0, pos1, rows0, rows1, rows2,
          isem, psem0, psem1,
          gsem0, gsem1, gsem2, ssem0, ssem1, ssem2):
    wid = lax.axis_index("s") * NC + lax.axis_index("c")
    s_base = wid * S_PER_W

    rows = (rows0, rows1, rows2)
    gsems = (gsem0, gsem1, gsem2)
    ssems = (ssem0, ssem1, ssem2)
    poss = (pos0, pos1)

    icopies = []
    for i in range(N_CHUNKS):
        k, b = divmod(i, BATCH)
        src = idx_hbm.at[pl.ds(b * SEQ_LEN + s_base + k * CHUNK, CHUNK)]
        icopies.append(pltpu.async_copy(src, idx_v.at[pl.ds(i * CHUNK, CHUNK)], isem))
    pcopies = [pltpu.async_copy(pos_hbm.at[pl.ds(s_base + 0 * CHUNK, CHUNK)], pos0, psem0),
               pltpu.async_copy(pos_hbm.at[pl.ds(s_base + 1 * CHUNK, CHUNK)], pos1, psem1)]

    def start_gather(i):
        return pltpu.async_copy(
            table_hbm.at[idx_v.at[pl.ds(i * CHUNK, CHUNK)]],
            rows[i % NBUF], gsems[i % NBUF])

    gathers = [None] * N_CHUNKS
    stores = [None] * N_CHUNKS
    icopies[0].wait()
    gathers[0] = start_gather(0)
    icopies[1].wait()
    gathers[1] = start_gather(1)
    for c in icopies[2:]:
        c.wait()
    pos_waited = [False, False]

    for i in range(N_CHUNKS):
        k, b = divmod(i, BATCH)
        buf = i % NBUF
        if i + 2 < N_CHUNKS:
            if i - 1 >= 0:
                stores[i - 1].wait()
            gathers[i + 2] = start_gather(i + 2)
        gathers[i].wait()
        if not pos_waited[k]:
            pcopies[k].wait()
            pos_waited[k] = True

        @plsc.parallel_loop(0, CHUNK, unroll=2)
        def add_row(r):
            for j in range(VECS_PER_ROW):
                sl = pl.ds(j * LANES, LANES)
                plsc.addupdate(rows[buf].at[r, sl], poss[k][r, sl])

        out_base = b * SEQ_LEN + s_base + k * CHUNK
        stores[i] = pltpu.async_copy(
            rows[buf], out_hbm.at[pl.ds(out_base, CHUNK)], ssems[buf])

    for i in range(N_CHUNKS - NBUF, N_CHUNKS):
        stores[i].wait()


@jax.jit
def _embed(inputs, embed_table, pos_table):
    mesh = plsc.VectorSubcoreMesh(core_axis_name="c", subcore_axis_name="s")
    return pl.kernel(
        _body,
        out_type=jax.ShapeDtypeStruct((BATCH * SEQ_LEN, D_MODEL), jnp.float32),
        mesh=mesh,
        scratch_types=[
            pltpu.VMEM((N_CHUNKS * CHUNK,), jnp.int32),
            pltpu.VMEM((CHUNK, D_MODEL), jnp.float32),
            pltpu.VMEM((CHUNK, D_MODEL), jnp.float32),
            pltpu.VMEM((CHUNK, D_MODEL), jnp.float32),
            pltpu.VMEM((CHUNK, D_MODEL), jnp.float32),
            pltpu.VMEM((CHUNK, D_MODEL), jnp.float32),
            pltpu.SemaphoreType.DMA,
            pltpu.SemaphoreType.DMA,
            pltpu.SemaphoreType.DMA,
            pltpu.SemaphoreType.DMA,
            pltpu.SemaphoreType.DMA,
            pltpu.SemaphoreType.DMA,
            pltpu.SemaphoreType.DMA,
            pltpu.SemaphoreType.DMA,
            pltpu.SemaphoreType.DMA,
        ],
    )(embed_table, inputs, pos_table)


def kernel(inputs, embed_table, pos_table):
    flat = _embed(inputs.reshape(-1).astype(jnp.int32), embed_table, pos_table)
    return flat.reshape(BATCH, SEQ_LEN, D_MODEL)


# speedup vs baseline: 1.0620x; 1.0620x over previous
import jax
import jax.numpy as jnp
from jax import lax
from jax.experimental import pallas as pl
from jax.experimental.pallas import tpu as pltpu
from jax.experimental.pallas import tpu_sc as plsc

VOCAB = 100000
D_MODEL = 768
SEQ_LEN = 2048
BATCH = 4

NC = 2
NS = 16
LANES = 16
NW = NC * NS

S_PER_W = SEQ_LEN // NW
CHUNK = 32
N_K = S_PER_W // CHUNK
N_CHUNKS = N_K * BATCH
NBUF = 3
VECS_PER_ROW = D_MODEL // LANES


def _body(table_hbm, idx_hbm, pos_hbm, out_hbm,
          idx_v, pos0, pos1, rows0, rows1, rows2,
          isem, psem0, psem1,
          gsem0, gsem1, gsem2, ssem0, ssem1, ssem2):
    wid = lax.axis_index("s") * NC + lax.axis_index("c")
    s_base = wid * S_PER_W

    rows = (rows0, rows1, rows2)
    gsems = (gsem0, gsem1, gsem2)
    ssems = (ssem0, ssem1, ssem2)
    poss = (pos0, pos1)

    icopies = []
    for i in range(N_CHUNKS):
        k, b = divmod(i, BATCH)
        src = idx_hbm.at[pl.ds(b * SEQ_LEN + s_base + k * CHUNK, CHUNK)]
        icopies.append(pltpu.async_copy(src, idx_v.at[pl.ds(i * CHUNK, CHUNK)], isem))
    pcopies = [pltpu.async_copy(pos_hbm.at[pl.ds(s_base + 0 * CHUNK, CHUNK)], pos0, psem0),
               pltpu.async_copy(pos_hbm.at[pl.ds(s_base + 1 * CHUNK, CHUNK)], pos1, psem1)]

    def start_gather(i):
        return pltpu.async_copy(
            table_hbm.at[idx_v.at[pl.ds(i * CHUNK, CHUNK)]],
            rows[i % NBUF], gsems[i % NBUF])

    gathers = [None] * N_CHUNKS
    stores = [None] * N_CHUNKS
    icopies[0].wait()
    gathers[0] = start_gather(0)
    icopies[1].wait()
    gathers[1] = start_gather(1)
    for c in icopies[2:]:
        c.wait()
    pos_waited = [False, False]

    for i in range(N_CHUNKS):
        k, b = divmod(i, BATCH)
        buf = i % NBUF
        if i + 2 < N_CHUNKS:
            if i - 1 >= 0:
                stores[i - 1].wait()
            gathers[i + 2] = start_gather(i + 2)
        gathers[i].wait()
        if not pos_waited[k]:
            pcopies[k].wait()
            pos_waited[k] = True

        def add_row(r, _):
            for j in range(VECS_PER_ROW):
                sl = pl.ds(j * LANES, LANES)
                plsc.addupdate(rows[buf].at[r, sl], poss[k][r, sl])
            return 0

        lax.fori_loop(0, CHUNK, add_row, 0, unroll=2)

        out_base = b * SEQ_LEN + s_base + k * CHUNK
        stores[i] = pltpu.async_copy(
            rows[buf], out_hbm.at[pl.ds(out_base, CHUNK)], ssems[buf])

    for i in range(N_CHUNKS - NBUF, N_CHUNKS):
        stores[i].wait()


@jax.jit
def _embed(inputs, embed_table, pos_table):
    mesh = plsc.VectorSubcoreMesh(core_axis_name="c", subcore_axis_name="s")
    return pl.kernel(
        _body,
        out_type=jax.ShapeDtypeStruct((BATCH * SEQ_LEN, D_MODEL), jnp.float32),
        mesh=mesh,
        scratch_types=[
            pltpu.VMEM((N_CHUNKS * CHUNK,), jnp.int32),
            pltpu.VMEM((CHUNK, D_MODEL), jnp.float32),
            pltpu.VMEM((CHUNK, D_MODEL), jnp.float32),
            pltpu.VMEM((CHUNK, D_MODEL), jnp.float32),
            pltpu.VMEM((CHUNK, D_MODEL), jnp.float32),
            pltpu.VMEM((CHUNK, D_MODEL), jnp.float32),
            pltpu.SemaphoreType.DMA,
            pltpu.SemaphoreType.DMA,
            pltpu.SemaphoreType.DMA,
            pltpu.SemaphoreType.DMA,
            pltpu.SemaphoreType.DMA,
            pltpu.SemaphoreType.DMA,
            pltpu.SemaphoreType.DMA,
            pltpu.SemaphoreType.DMA,
            pltpu.SemaphoreType.DMA,
        ],
    )(embed_table, inputs, pos_table)


def kernel(inputs, embed_table, pos_table):
    flat = _embed(inputs.reshape(-1).astype(jnp.int32), embed_table, pos_table)
    return flat.reshape(BATCH, SEQ_LEN, D_MODEL)
